# baseline (speedup 1.0000x reference)
import functools

import jax
import jax.numpy as jnp
from jax import lax
from jax.experimental import pallas as pl
from jax.experimental.pallas import tpu as pltpu
from jax.experimental.pallas import tpu_sc as plsc

B, S, D = 4, 8192, 1024
NC, NS = 2, 16
NW = NC * NS
S_PER_W = S // NW
CHUNK = 32
N_CHUNKS = S_PER_W // CHUNK
CELEMS = CHUNK * D
LANES = 16


def _sc_body(x_hbm, pe_hbm, out_hbm, pe_v, x_v, sem):
    wid = lax.axis_index("s") * NC + lax.axis_index("c")
    base = wid * (S_PER_W * D)

    def chunk_body(c, carry):
        off = pl.multiple_of(base + c * CELEMS, 8)
        pltpu.sync_copy(pe_hbm.at[pl.ds(off, CELEMS)], pe_v)

        def batch_body(b, carry2):
            xoff = pl.multiple_of(b * (S * D) + off, 8)
            pltpu.sync_copy(x_hbm.at[pl.ds(xoff, CELEMS)], x_v)

            def add_body(i, carry3):
                j = pl.multiple_of(i * LANES, 8)
                x_v[pl.ds(j, LANES)] = x_v[pl.ds(j, LANES)] + pe_v[pl.ds(j, LANES)]
                return carry3

            lax.fori_loop(0, CELEMS // LANES, add_body, 0)
            pltpu.sync_copy(x_v, out_hbm.at[pl.ds(xoff, CELEMS)])
            return carry2

        lax.fori_loop(0, B, batch_body, 0)
        return carry

    lax.fori_loop(0, N_CHUNKS, chunk_body, 0)


@jax.jit
def kernel(x, pe_table):
    mesh = plsc.VectorSubcoreMesh(core_axis_name="c", subcore_axis_name="s")
    k = functools.partial(
        pl.kernel,
        mesh=mesh,
        out_type=jax.ShapeDtypeStruct((B * S * D,), jnp.float32),
        scratch_types=[
            pltpu.VMEM((CELEMS,), jnp.float32),
            pltpu.VMEM((CELEMS,), jnp.float32),
            pltpu.SemaphoreType.DMA,
        ],
    )(_sc_body)
    out_flat = k(x.reshape(-1), pe_table.reshape(-1))
    return out_flat.reshape(B, S, D)


# speedup vs baseline: 1.8301x; 1.8301x over previous
import functools

import jax
import jax.numpy as jnp
from jax import lax
from jax.experimental import pallas as pl
from jax.experimental.pallas import tpu as pltpu
from jax.experimental.pallas import tpu_sc as plsc

B, S, D = 4, 8192, 1024
NC, NS = 2, 16
NW = NC * NS
S_PER_W = S // NW
CHUNK = 8
N_CHUNKS = S_PER_W // CHUNK
CB = CHUNK * D
LANES = 16


def _sc_body(x_hbm, pe_hbm, out_hbm,
             peb0, peb1,
             xb00, xb01, xb02, xb03,
             xb10, xb11, xb12, xb13,
             pe_sem0, pe_sem1, in_sem0, in_sem1, out_sem):
    peb = (peb0, peb1)
    xb = ((xb00, xb01, xb02, xb03), (xb10, xb11, xb12, xb13))
    pe_sem = (pe_sem0, pe_sem1)
    in_sem = (in_sem0, in_sem1)

    wid = lax.axis_index("s") * NC + lax.axis_index("c")
    base = wid * (S_PER_W * D)

    def pe_off(c):
        return pl.multiple_of(base + c * CB, 8)

    def x_off(c, b):
        return pl.multiple_of(b * (S * D) + base + c * CB, 8)

    def issue_pe(c, p):
        pltpu.async_copy(pe_hbm.at[pl.ds(pe_off(c), CB)], peb[p], pe_sem[p])

    def issue_in(c, p):
        for b in range(B):
            pltpu.async_copy(x_hbm.at[pl.ds(x_off(c, b), CB)], xb[p][b],
                             in_sem[p])

    def wait_pe(p):
        pltpu.make_async_copy(pe_hbm.at[pl.ds(0, CB)], peb[p], pe_sem[p]).wait()

    def wait_in(p):
        for b in range(B):
            pltpu.make_async_copy(x_hbm.at[pl.ds(0, CB)], xb[p][b],
                                  in_sem[p]).wait()

    def drain_outs():
        for b in range(B):
            pltpu.make_async_copy(x_hbm.at[pl.ds(0, CB)], xb[0][b],
                                  out_sem).wait()

    def chunk_step(c, p):
        wait_pe(p)

        @pl.when(c + 1 < N_CHUNKS)
        def _():
            issue_pe(c + 1, 1 - p)

        wait_in(p)

        @pl.when(c > 0)
        def _():
            drain_outs()

        @pl.when(c + 1 < N_CHUNKS)
        def _():
            issue_in(c + 1, 1 - p)

        for b in range(B):
            buf = xb[p][b]
            pbuf = peb[p]

            @plsc.parallel_loop(0, CB, LANES, unroll=8)
            def _(i, buf=buf, pbuf=pbuf):
                j = pl.multiple_of(i, 8)
                buf[pl.ds(j, LANES)] = buf[pl.ds(j, LANES)] + pbuf[pl.ds(j, LANES)]

            pltpu.async_copy(buf, out_hbm.at[pl.ds(x_off(c, b), CB)], out_sem)

    issue_pe(0, 0)
    issue_in(0, 0)

    def loop_body(t, carry):
        chunk_step(2 * t, 0)
        chunk_step(2 * t + 1, 1)
        return carry

    lax.fori_loop(0, N_CHUNKS // 2, loop_body, 0)
    drain_outs()


@jax.jit
def kernel(x, pe_table):
    mesh = plsc.VectorSubcoreMesh(core_axis_name="c", subcore_axis_name="s")
    k = functools.partial(
        pl.kernel,
        mesh=mesh,
        out_type=jax.ShapeDtypeStruct((B * S * D,), jnp.float32),
        scratch_types=(
            [pltpu.VMEM((CB,), jnp.float32)] * 2
            + [pltpu.VMEM((CB,), jnp.float32)] * 8
            + [pltpu.SemaphoreType.DMA] * 5
        ),
    )(_sc_body)
    out_flat = k(x.reshape(-1), pe_table.reshape(-1))
    return out_flat.reshape(B, S, D)


# speedup vs baseline: 5.5866x; 3.0526x over previous
import functools

import jax
import jax.numpy as jnp
from jax import lax
from jax.experimental import pallas as pl
from jax.experimental.pallas import tpu as pltpu
from jax.experimental.pallas import tpu_sc as plsc

B, S, D = 4, 8192, 1024
NC, NS = 2, 16
NW = NC * NS
S_PER_W = S // NW
CHUNK = 8
N_CHUNKS = S_PER_W // CHUNK
LANES = 16


def _sc_body(x_hbm, pe_hbm, out_hbm,
             peb0, peb1,
             xb00, xb01, xb02, xb03,
             xb10, xb11, xb12, xb13,
             pe_sem0, pe_sem1, in_sem0, in_sem1, out_sem):
    peb = (peb0, peb1)
    xb = ((xb00, xb01, xb02, xb03), (xb10, xb11, xb12, xb13))
    pe_sem = (pe_sem0, pe_sem1)
    in_sem = (in_sem0, in_sem1)

    wid = lax.axis_index("s") * NC + lax.axis_index("c")
    base = wid * S_PER_W

    def pe_row(c):
        return pl.multiple_of(base + c * CHUNK, 8)

    def x_row(c, b):
        return pl.multiple_of(b * S + base + c * CHUNK, 8)

    def issue_pe(c, p):
        pltpu.async_copy(pe_hbm.at[pl.ds(pe_row(c), CHUNK)], peb[p], pe_sem[p])

    def issue_in(c, p):
        for b in range(B):
            pltpu.async_copy(x_hbm.at[pl.ds(x_row(c, b), CHUNK)], xb[p][b],
                             in_sem[p])

    def wait_pe(p):
        pltpu.make_async_copy(pe_hbm.at[pl.ds(0, CHUNK)], peb[p],
                              pe_sem[p]).wait()

    def wait_in(p):
        for b in range(B):
            pltpu.make_async_copy(x_hbm.at[pl.ds(0, CHUNK)], xb[p][b],
                                  in_sem[p]).wait()

    def drain_outs():
        for b in range(B):
            pltpu.make_async_copy(x_hbm.at[pl.ds(0, CHUNK)], xb[0][b],
                                  out_sem).wait()

    def chunk_step(c, p):
        wait_pe(p)

        @pl.when(c + 1 < N_CHUNKS)
        def _():
            issue_pe(c + 1, 1 - p)

        wait_in(p)

        @pl.when(c > 0)
        def _():
            drain_outs()

        @pl.when(c + 1 < N_CHUNKS)
        def _():
            issue_in(c + 1, 1 - p)

        bufs = xb[p]
        pbuf = peb[p]

        for r in range(CHUNK):
            @plsc.parallel_loop(0, D, LANES, unroll=4)
            def _(i, r=r):
                j = pl.multiple_of(i, 8)
                pe_slice = pbuf[r, pl.ds(j, LANES)]
                for b in range(B):
                    bufs[b][r, pl.ds(j, LANES)] = (
                        bufs[b][r, pl.ds(j, LANES)] + pe_slice)

        for b in range(B):
            pltpu.async_copy(bufs[b], out_hbm.at[pl.ds(x_row(c, b), CHUNK)],
                             out_sem)

    issue_pe(0, 0)
    issue_in(0, 0)

    def loop_body(t, carry):
        chunk_step(2 * t, 0)
        chunk_step(2 * t + 1, 1)
        return carry

    lax.fori_loop(0, N_CHUNKS // 2, loop_body, 0)
    drain_outs()


@jax.jit
def kernel(x, pe_table):
    mesh = plsc.VectorSubcoreMesh(core_axis_name="c", subcore_axis_name="s")
    k = functools.partial(
        pl.kernel,
        mesh=mesh,
        out_type=jax.ShapeDtypeStruct((B * S, D), jnp.float32),
        scratch_types=(
            [pltpu.VMEM((CHUNK, D), jnp.float32)] * 2
            + [pltpu.VMEM((CHUNK, D), jnp.float32)] * 8
            + [pltpu.SemaphoreType.DMA] * 5
        ),
    )(_sc_body)
    out2d = k(x.reshape(B * S, D), pe_table)
    return out2d.reshape(B, S, D)
